# baseline (speedup 1.0000x reference)
import functools

import jax
import jax.numpy as jnp
from jax import lax
from jax.experimental import pallas as pl
from jax.experimental.pallas import tpu as pltpu
from jax.experimental.pallas import tpu_sc as plsc

NUM_NODES = 100000
FEAT_DIM = 128
LANES = 16

RB = 12288
NBK = 9


def _sc_gather_body(node_hbm, ctxt_hbm, out_hbm, node_s, sem):
    c = lax.axis_index("c")

    @pl.when(c == 0)
    def _():
        pltpu.sync_copy(node_hbm, node_s)
        n = node_s[0]
        pltpu.async_copy(ctxt_hbm.at[pl.ds(n, 1), :], out_hbm, sem).wait()


@functools.cache
def _build_sc_gather():
    return functools.partial(
        pl.kernel,
        mesh=plsc.ScalarSubcoreMesh(axis_name="c", num_cores=2),
        out_type=jax.ShapeDtypeStruct((1, FEAT_DIM), jnp.float32),
        scratch_types=[
            pltpu.SMEM((LANES,), jnp.int32),
            pltpu.SemaphoreType.DMA,
        ],
    )(_sc_gather_body)


def _fused_body(z1_ref, z_ref, out_ref, z2s, acc):
    i = pl.program_id(0)

    @pl.when(i < NBK)
    def _():
        s_row = lax.dot_general(
            z1_ref[...], z_ref[...], (((1,), (1,)), ((), ())),
            preferred_element_type=jnp.float32,
        )
        gidx = lax.broadcasted_iota(jnp.int32, (1, RB), 1) + i * RB
        s_row = jnp.where(gidx < NUM_NODES, s_row, -jnp.inf)
        z2s[pl.ds(i, 1), :] = s_row
        bm = jnp.max(s_row)
        m_prev = jnp.where(i == 0, -jnp.inf, acc[0])
        s_prev = jnp.where(i == 0, 0.0, acc[1])
        m_new = jnp.maximum(m_prev, bm)
        s_new = s_prev * jnp.exp(m_prev - m_new) + jnp.sum(jnp.exp(s_row - m_new))
        acc[0] = m_new
        acc[1] = s_new

    @pl.when(i == NBK)
    def _():
        lse = acc[0] + jnp.log(acc[1])
        flat = jnp.concatenate([z2s[pl.ds(j, 1), :] for j in range(NBK)], axis=1)
        out_ref[...] = (flat[:, :NUM_NODES] - lse).reshape(NUM_NODES)


def _fused(z1, z):
    return pl.pallas_call(
        _fused_body,
        grid=(NBK + 1,),
        in_specs=[
            pl.BlockSpec((1, FEAT_DIM), lambda i: (0, 0)),
            pl.BlockSpec((RB, FEAT_DIM), lambda i: (jnp.minimum(i, NBK - 1), 0)),
        ],
        out_specs=pl.BlockSpec((NUM_NODES,), lambda i: (0,)),
        out_shape=jax.ShapeDtypeStruct((NUM_NODES,), jnp.float32),
        scratch_shapes=[
            pltpu.VMEM((NBK, RB), jnp.float32),
            pltpu.SMEM((2,), jnp.float32),
        ],
    )(z1, z)


def kernel(node, z, context_embds):
    node16 = jnp.full((LANES,), node, jnp.int32)
    ctxt = context_embds.T
    z1 = _build_sc_gather()(node16, ctxt)
    return _fused(z1, z)


# speedup vs baseline: 1.0684x; 1.0684x over previous
import functools

import jax
import jax.numpy as jnp
from jax import lax
from jax.experimental import pallas as pl
from jax.experimental.pallas import tpu as pltpu
from jax.experimental.pallas import tpu_sc as plsc

NUM_NODES = 100000
FEAT_DIM = 128
LANES = 16

RB = 20480
NBK = 5


def _sc_gather_body(node_hbm, ctxt_hbm, out_hbm, node_s, sem):
    c = lax.axis_index("c")

    @pl.when(c == 0)
    def _():
        pltpu.sync_copy(node_hbm, node_s)
        n = node_s[0]
        pltpu.async_copy(ctxt_hbm.at[pl.ds(n, 1), :], out_hbm, sem).wait()


@functools.cache
def _build_sc_gather():
    return functools.partial(
        pl.kernel,
        mesh=plsc.ScalarSubcoreMesh(axis_name="c", num_cores=2),
        out_type=jax.ShapeDtypeStruct((1, FEAT_DIM), jnp.float32),
        scratch_types=[
            pltpu.SMEM((LANES,), jnp.int32),
            pltpu.SemaphoreType.DMA,
        ],
    )(_sc_gather_body)


def _fused_body(z1_ref, z_ref, out_ref, z2s, acc):
    i = pl.program_id(0)

    @pl.when(i < NBK)
    def _():
        s_row = lax.dot_general(
            z1_ref[...], z_ref[...], (((1,), (1,)), ((), ())),
            preferred_element_type=jnp.float32,
        )
        gidx = lax.broadcasted_iota(jnp.int32, (1, RB), 1) + i * RB
        s_row = jnp.where(gidx < NUM_NODES, s_row, -jnp.inf)
        z2s[pl.ds(i, 1), :] = s_row
        bm = jnp.max(s_row)
        m_prev = jnp.where(i == 0, -jnp.inf, acc[0])
        s_prev = jnp.where(i == 0, 0.0, acc[1])
        m_new = jnp.maximum(m_prev, bm)
        s_new = s_prev * jnp.exp(m_prev - m_new) + jnp.sum(jnp.exp(s_row - m_new))
        acc[0] = m_new
        acc[1] = s_new

    @pl.when(i == NBK)
    def _():
        lse = acc[0] + jnp.log(acc[1])
        flat = jnp.concatenate([z2s[pl.ds(j, 1), :] for j in range(NBK)], axis=1)
        out_ref[...] = (flat[:, :NUM_NODES] - lse).reshape(NUM_NODES)


def _fused(z1, z):
    return pl.pallas_call(
        _fused_body,
        grid=(NBK + 1,),
        in_specs=[
            pl.BlockSpec((1, FEAT_DIM), lambda i: (0, 0)),
            pl.BlockSpec((RB, FEAT_DIM), lambda i: (jnp.minimum(i, NBK - 1), 0)),
        ],
        out_specs=pl.BlockSpec((NUM_NODES,), lambda i: (0,)),
        out_shape=jax.ShapeDtypeStruct((NUM_NODES,), jnp.float32),
        scratch_shapes=[
            pltpu.VMEM((NBK, RB), jnp.float32),
            pltpu.SMEM((2,), jnp.float32),
        ],
    )(z1, z)


def kernel(node, z, context_embds):
    node16 = jnp.full((LANES,), node, jnp.int32)
    ctxt = context_embds.T
    z1 = _build_sc_gather()(node16, ctxt)
    return _fused(z1, z)
